# Initial kernel scaffold; baseline (speedup 1.0000x reference)
import jax
import jax.numpy as jnp
from jax.experimental import pallas as pl


def kernel(emb, edge_index, edge_weight):
    raise NotImplementedError("write your pallas kernel here")


# speedup vs baseline: 6.5872x; 6.5872x over previous
import functools

import jax
import jax.numpy as jnp
from jax import lax
from jax.experimental import pallas as pl
from jax.experimental.pallas import tpu as pltpu
from jax.experimental.pallas import tpu_sc as plsc

N = 100000
D = 32
NC = 2
NS = 16
H = N // NC
CH = 512
G = CH // 128
NCH = 196
TPS = CH * NCH
E_PAD = TPS * NS


def _layer_body(scale, x_hbm, s_hbm, src_hbm, dst_hbm, w_hbm, xo_hbm, so_hbm,
                acc, sidx, dstv, dloc, wv, rows, gsem):
    c = lax.axis_index("c")
    sid = lax.axis_index("s")
    base = c * H
    z16 = jnp.zeros((16,), jnp.float32)

    def zbody(e, carry):
        rows[e, pl.ds(0, 16)] = z16
        rows[e, pl.ds(16, 16)] = z16
        return carry
    lax.fori_loop(0, CH, zbody, 0)
    for t in range(7):
        b = sid + 16 * t
        @pl.when(b <= 96)
        def _():
            pltpu.sync_copy(rows.at[pl.ds(0, CH)], acc.at[pl.ds(b * CH, CH)])
    @pl.when(sid == 1)
    def _():
        pltpu.sync_copy(rows.at[pl.ds(0, 352)], acc.at[pl.ds(97 * CH, 352)])
    plsc.subcore_barrier()

    trow = sid * (TPS // 128)
    toff = sid * TPS

    def chunk(k, carry):
        pltpu.sync_copy(src_hbm.at[pl.ds(trow + k * G, G)], sidx)
        pltpu.sync_copy(dst_hbm.at[pl.ds(trow + k * G, G)], dstv)
        pltpu.sync_copy(w_hbm.at[pl.ds(toff + k * CH, CH)], wv)
        cps = [pltpu.make_async_copy(x_hbm.at[sidx.at[r]],
                                     rows.at[pl.ds(r * 128, 128)], gsem)
               for r in range(G)]
        for cp in cps:
            cp.start()

        def dmap(j, carry2):
            r = j // 8
            q = (j % 8) * 16
            d = dstv[r, pl.ds(q, 16)]
            loc = d - base
            ok = (loc >= 0) & (loc < H)
            dloc[r, pl.ds(q, 16)] = jnp.where(ok, loc, H)
            return carry2
        lax.fori_loop(0, CH // 16, dmap, 0)

        for cp in cps:
            cp.wait()

        def wmul(j, carry2):
            wgrp = wv[pl.ds(j * 16, 16)]
            e0 = j * 16
            for i in range(16):
                w = wgrp[i]
                rows[e0 + i, pl.ds(0, 16)] = rows[e0 + i, pl.ds(0, 16)] * w
                rows[e0 + i, pl.ds(16, 16)] = rows[e0 + i, pl.ds(16, 16)] * w
            return carry2
        lax.fori_loop(0, CH // 16, wmul, 0)

        for r in range(G):
            pltpu.sync_copy(rows.at[pl.ds(r * 128, 128)], acc.at[dloc.at[r]], add=True)
        return carry
    lax.fori_loop(0, NCH, chunk, 0)
    plsc.subcore_barrier()

    WB = CH // 2

    def wout(o, n):
        pltpu.sync_copy(acc.at[pl.ds(o, n)], rows.at[pl.ds(0, n)])
        pltpu.sync_copy(s_hbm.at[pl.ds(base + o, n)], rows.at[pl.ds(WB, n)])

        def sadd(e, carry):
            a0 = rows[e, pl.ds(0, 16)] + rows[WB + e, pl.ds(0, 16)]
            a1 = rows[e, pl.ds(16, 16)] + rows[WB + e, pl.ds(16, 16)]
            if scale != 1.0:
                a0 = a0 * scale
                a1 = a1 * scale
            rows[WB + e, pl.ds(0, 16)] = a0
            rows[WB + e, pl.ds(16, 16)] = a1
            return carry
        lax.fori_loop(0, n, sadd, 0)
        pltpu.sync_copy(rows.at[pl.ds(0, n)], xo_hbm.at[pl.ds(base + o, n)])
        pltpu.sync_copy(rows.at[pl.ds(WB, n)], so_hbm.at[pl.ds(base + o, n)])

    for t in range(13):
        b = sid + 16 * t
        @pl.when(b <= 194)
        def _():
            wout(b * WB, WB)
    @pl.when(sid == 3)
    def _():
        wout(195 * WB, 80)


def _make_layer(scale):
    return pl.kernel(
        functools.partial(_layer_body, scale),
        out_type=(jax.ShapeDtypeStruct((N, D), jnp.float32),
                  jax.ShapeDtypeStruct((N, D), jnp.float32)),
        mesh=plsc.VectorSubcoreMesh(core_axis_name="c", subcore_axis_name="s"),
        compiler_params=pltpu.CompilerParams(use_tc_tiling_on_sc=False),
        scratch_types=[
            pltpu.VMEM_SHARED((H + 16, D), jnp.float32),
            pltpu.VMEM((G, 128), jnp.int32),
            pltpu.VMEM((G, 128), jnp.int32),
            pltpu.VMEM((G, 128), jnp.int32),
            pltpu.VMEM((CH,), jnp.float32),
            pltpu.VMEM((CH, D), jnp.float32),
            pltpu.SemaphoreType.DMA,
        ],
    )


_layer_mid = _make_layer(1.0)
_layer_last = _make_layer(0.25)


def kernel(emb, edge_index, edge_weight):
    e = edge_index.shape[1]
    pad = E_PAD - e
    src = jnp.concatenate([edge_index[0], jnp.zeros((pad,), jnp.int32)]).reshape(-1, 128)
    dst = jnp.concatenate([edge_index[1], jnp.zeros((pad,), jnp.int32)]).reshape(-1, 128)
    w = jnp.concatenate([edge_weight, jnp.zeros((pad,), jnp.float32)])
    x = emb
    s = emb
    x, s = _layer_mid(x, s, src, dst, w)
    x, s = _layer_mid(x, s, src, dst, w)
    x, s = _layer_last(x, s, src, dst, w)
    return s
